# baseline (speedup 1.0000x reference)
import functools

import jax
import jax.numpy as jnp
from jax import lax
from jax.experimental import pallas as pl
from jax.experimental.pallas import tpu as pltpu
from jax.experimental.pallas import tpu_sc as plsc

EMB_DIM = 64
N_OUTER = 16384
N_INNER = 50
B_TOTAL = N_OUTER * N_INNER
NROWS = 1000000

_info = plsc.get_sparse_core_info()
_NC, _NS = _info.num_cores, _info.num_subcores
_NW = _NC * _NS

_mesh = plsc.VectorSubcoreMesh(core_axis_name="c", subcore_axis_name="s")


def _transpose16(regs):
    lanes = lax.iota(jnp.int32, 16)
    for s in (1, 2, 4, 8):
        perm = jnp.bitwise_xor(lanes, s)
        hi = (lanes & s) != 0
        new = list(regs)
        for p in range(16):
            if p & s:
                continue
            q = p | s
            top, bot = regs[p], regs[q]
            pt = top.at[perm].get(mode="promise_in_bounds")
            pb = bot.at[perm].get(mode="promise_in_bounds")
            new[p] = jnp.where(hi, pb, top)
            new[q] = jnp.where(hi, bot, pt)
        regs = new
    return regs


_K1_BLOCKS = NROWS // 128
_K1_PER_W = (_K1_BLOCKS + _NW - 1) // _NW


@functools.partial(
    pl.kernel,
    mesh=_mesh,
    out_type=jax.ShapeDtypeStruct((NROWS // 2, 128), jnp.float32),
    scratch_types=[
        pltpu.VMEM((2, EMB_DIM, 128), jnp.float32),
        pltpu.VMEM((2, EMB_DIM, 128), jnp.float32),
        pltpu.VMEM((EMB_DIM, 64), jnp.float32),
        pltpu.SemaphoreType.DMA((2,)),
        pltpu.SemaphoreType.DMA((2,)),
    ],
    compiler_params=pltpu.CompilerParams(use_tc_tiling_on_sc=True),
)
def _pack_kernel(tt_hbm, tailt_hbm, out_hbm, in_v, t_v, tail_v, gsem, wsem):
    wid = lax.axis_index("s") * _NC + lax.axis_index("c")

    def start_in(c, b):
        pltpu.async_copy(
            tt_hbm.at[:, pl.ds(c * 128, 128)], in_v.at[b], gsem.at[b])

    def wait_in(b):
        pltpu.make_async_copy(
            tt_hbm.at[:, pl.ds(0, 128)], in_v.at[b], gsem.at[b]).wait()

    def transpose_block(rows, tref, nr):

        def rblock(rB, _):
            for dB in range(4):
                regs = [rows[dB * 16 + k, pl.ds(rB * 16, 16)]
                        for k in range(16)]
                tregs = _transpose16(regs)
                for k in range(16):
                    tref[rB * 8 + k // 2,
                         pl.ds((k % 2) * 64 + dB * 16, 16)] = tregs[k]
            return ()

        lax.fori_loop(0, nr, rblock, ())

    def start_out(c, b):
        pltpu.async_copy(
            t_v.at[b], out_hbm.at[pl.ds(c * 64, 64)], wsem.at[b])

    def wait_out(b):
        pltpu.make_async_copy(
            t_v.at[b], out_hbm.at[pl.ds(0, 64)], wsem.at[b]).wait()

    start_in(wid, 0)

    def step(m, _):
        for u in range(2):
            s = m * 2 + u
            b = u
            c = s * _NW + wid
            c1 = (s + 1) * _NW + wid

            @pl.when(c1 < _K1_BLOCKS)
            def _():
                start_in(c1, 1 - b)

            @pl.when(c < _K1_BLOCKS)
            def _():
                wait_in(b)

                @pl.when(s >= 2)
                def _():
                    wait_out(b)

                transpose_block(in_v.at[b], t_v.at[b], 8)
                start_out(c, b)

        return ()

    lax.fori_loop(0, (_K1_PER_W + 1) // 2, step, ())
    wait_out(0)
    wait_out(1)

    @pl.when(wid == 0)
    def _():
        pltpu.async_copy(tailt_hbm, tail_v, gsem.at[0])
        pltpu.make_async_copy(tailt_hbm, tail_v, gsem.at[0]).wait()
        transpose_block(tail_v, t_v.at[0], 4)
        pltpu.async_copy(
            t_v.at[0].at[pl.ds(0, 32)],
            out_hbm.at[pl.ds(_K1_BLOCKS * 64, 32)], wsem.at[0])
        pltpu.make_async_copy(
            t_v.at[0].at[pl.ds(0, 32)],
            out_hbm.at[pl.ds(0, 32)], wsem.at[0]).wait()


_TILES = 2
_GROUP = _TILES * 128
_NGROUPS = B_TOTAL // _GROUP
_G_PER_W = _NGROUPS // _NW
_TPJ = N_OUTER // _GROUP
_NBUF = 2


@functools.partial(
    pl.kernel,
    mesh=_mesh,
    out_type=jax.ShapeDtypeStruct((N_INNER, 8, 128, 8, 128), jnp.float32),
    scratch_types=[
        pltpu.VMEM((_G_PER_W * _GROUP * 2,), jnp.int32),
        pltpu.VMEM((_NBUF, _GROUP * 2, EMB_DIM // 2), jnp.float32),
        pltpu.VMEM((_NBUF, 8, _TILES, 8, 128), jnp.float32),
        pltpu.SemaphoreType.DMA,
        pltpu.SemaphoreType.DMA((_NBUF,)),
        pltpu.SemaphoreType.DMA((_NBUF,)),
    ],
    compiler_params=pltpu.CompilerParams(use_tc_tiling_on_sc=False),
)
def _gather_kernel(idx2_hbm, pk_hbm, out_hbm, idx_v, rows_v, t_v, isem,
                   gsem, wsem):
    wid = lax.axis_index("s") * _NC + lax.axis_index("c")
    g0 = wid * _G_PER_W
    n0 = g0 * _GROUP * 2
    nw = _G_PER_W * _GROUP * 2
    pltpu.async_copy(idx2_hbm.at[pl.ds(n0, nw)], idx_v, isem)
    pltpu.make_async_copy(idx2_hbm.at[pl.ds(0, nw)], idx_v, isem).wait()

    def start_gather(s, b):
        pltpu.async_copy(
            pk_hbm.at[idx_v.at[pl.ds(s * _GROUP * 2, _GROUP * 2)]],
            rows_v.at[b], gsem.at[b])

    def wait_gather(b):
        pltpu.make_async_copy(
            pk_hbm.at[idx_v.at[pl.ds(0, _GROUP * 2)]],
            rows_v.at[b], gsem.at[b]).wait()

    def transpose(b):
        rows = rows_v.at[b]
        tref = t_v.at[b]

        def iblock(ib, _):
            row0 = ib * 16
            for d0 in range(0, EMB_DIM, 16):
                h = d0 // 32
                c0 = d0 % 32
                regs = [rows[(row0 + p) * 2 + h, pl.ds(c0, 16)]
                        for p in range(16)]
                tregs = _transpose16(regs)
                for p in range(16):
                    d = d0 + p
                    tref[d // 8, ib // 8, d % 8,
                         pl.ds((ib % 8) * 16, 16)] = tregs[p]
            return ()

        lax.fori_loop(0, _GROUP // 16, iblock, ())

    def start_wb(s, b):
        g = g0 + s
        pltpu.async_copy(
            t_v.at[b],
            out_hbm.at[g // _TPJ, :, pl.ds((g % _TPJ) * _TILES, _TILES)],
            wsem.at[b])

    def wait_wb(b):
        pltpu.make_async_copy(
            pk_hbm.at[pl.ds(0, _GROUP * 2)], rows_v.at[b],
            wsem.at[b]).wait()

    start_gather(0, 0)

    def step(m, _):
        for u in range(_NBUF):
            s = m * _NBUF + u
            b = u
            b1 = (u + 1) % _NBUF

            @pl.when(s + 1 < _G_PER_W)
            def _():
                start_gather(s + 1, b1)

            @pl.when(s < _G_PER_W)
            def _():
                wait_gather(b)

                @pl.when(s >= _NBUF)
                def _():
                    wait_wb(b)

                transpose(b)
                start_wb(s, b)

        return ()

    nsteps = (_G_PER_W + _NBUF - 1) // _NBUF
    lax.fori_loop(0, nsteps, step, ())

    for b in range(_NBUF):
        wait_wb(b)


def kernel(x, table):
    tt = table.T
    packed = _pack_kernel(tt, lax.slice(tt, (0, NROWS - 64), (EMB_DIM, NROWS)))
    pk2 = packed.reshape(2 * NROWS, EMB_DIM // 2)
    xt = x.T.astype(jnp.int32).reshape(-1)
    idx2 = jnp.stack([xt * 2, xt * 2 + 1], axis=-1).reshape(-1)
    out5 = _gather_kernel(idx2, pk2)
    return out5.transpose(2, 4, 0, 1, 3).reshape(N_OUTER, N_INNER, EMB_DIM)


# speedup vs baseline: 2.5103x; 2.5103x over previous
import functools

import jax
import jax.numpy as jnp
from jax import lax
from jax.experimental import pallas as pl
from jax.experimental.pallas import tpu as pltpu
from jax.experimental.pallas import tpu_sc as plsc

EMB_DIM = 64
N_OUTER = 16384
N_INNER = 50
B_TOTAL = N_OUTER * N_INNER
NROWS = 1000000

_info = plsc.get_sparse_core_info()
_NC, _NS = _info.num_cores, _info.num_subcores
_NW = _NC * _NS

_mesh = plsc.VectorSubcoreMesh(core_axis_name="c", subcore_axis_name="s")


def _transpose16(regs):
    lanes = lax.iota(jnp.int32, 16)
    for s in (1, 2, 4, 8):
        perm = jnp.bitwise_xor(lanes, s)
        hi = (lanes & s) != 0
        new = list(regs)
        for p in range(16):
            if p & s:
                continue
            q = p | s
            top, bot = regs[p], regs[q]
            pt = top.at[perm].get(mode="promise_in_bounds")
            pb = bot.at[perm].get(mode="promise_in_bounds")
            new[p] = jnp.where(hi, pb, top)
            new[q] = jnp.where(hi, bot, pt)
        regs = new
    return regs


_K1_BLOCKS = NROWS // 128
_K1_PER_W = (_K1_BLOCKS + _NW - 1) // _NW


@functools.partial(
    pl.kernel,
    mesh=_mesh,
    out_type=jax.ShapeDtypeStruct((NROWS // 2, 128), jnp.float32),
    scratch_types=[
        pltpu.VMEM((2, EMB_DIM, 128), jnp.float32),
        pltpu.VMEM((2, EMB_DIM, 128), jnp.float32),
        pltpu.VMEM((EMB_DIM, 64), jnp.float32),
        pltpu.SemaphoreType.DMA((2,)),
        pltpu.SemaphoreType.DMA((2,)),
    ],
    compiler_params=pltpu.CompilerParams(use_tc_tiling_on_sc=True),
)
def _pack_kernel(tt_hbm, tailt_hbm, out_hbm, in_v, t_v, tail_v, gsem, wsem):
    wid = lax.axis_index("s") * _NC + lax.axis_index("c")

    def start_in(c, b):
        pltpu.async_copy(
            tt_hbm.at[:, pl.ds(c * 128, 128)], in_v.at[b], gsem.at[b])

    def wait_in(b):
        pltpu.make_async_copy(
            tt_hbm.at[:, pl.ds(0, 128)], in_v.at[b], gsem.at[b]).wait()

    def transpose_block(rows, tref, nr):

        def rblock(rB, _):
            for dB in range(4):
                regs = [rows[dB * 16 + k, pl.ds(rB * 16, 16)]
                        for k in range(16)]
                tregs = _transpose16(regs)
                for k in range(16):
                    tref[rB * 8 + k // 2,
                         pl.ds((k % 2) * 64 + dB * 16, 16)] = tregs[k]
            return ()

        lax.fori_loop(0, nr, rblock, ())

    def start_out(c, b):
        pltpu.async_copy(
            t_v.at[b], out_hbm.at[pl.ds(c * 64, 64)], wsem.at[b])

    def wait_out(b):
        pltpu.make_async_copy(
            t_v.at[b], out_hbm.at[pl.ds(0, 64)], wsem.at[b]).wait()

    start_in(wid, 0)

    def step(m, _):
        for u in range(2):
            s = m * 2 + u
            b = u
            c = s * _NW + wid
            c1 = (s + 1) * _NW + wid

            @pl.when(c1 < _K1_BLOCKS)
            def _():
                start_in(c1, 1 - b)

            @pl.when(c < _K1_BLOCKS)
            def _():
                wait_in(b)

                @pl.when(s >= 2)
                def _():
                    wait_out(b)

                transpose_block(in_v.at[b], t_v.at[b], 8)
                start_out(c, b)

        return ()

    lax.fori_loop(0, (_K1_PER_W + 1) // 2, step, ())
    wait_out(0)
    wait_out(1)

    @pl.when(wid == 0)
    def _():
        pltpu.async_copy(tailt_hbm, tail_v, gsem.at[0])
        pltpu.make_async_copy(tailt_hbm, tail_v, gsem.at[0]).wait()
        transpose_block(tail_v, t_v.at[0], 4)
        pltpu.async_copy(
            t_v.at[0].at[pl.ds(0, 32)],
            out_hbm.at[pl.ds(_K1_BLOCKS * 64, 32)], wsem.at[0])
        pltpu.make_async_copy(
            t_v.at[0].at[pl.ds(0, 32)],
            out_hbm.at[pl.ds(0, 32)], wsem.at[0]).wait()


_TILES = 2
_GROUP = _TILES * 128
_NGROUPS = B_TOTAL // _GROUP
_G_PER_W = _NGROUPS // _NW
_TPJ = N_OUTER // _GROUP
_NBUF = 2


@functools.partial(
    pl.kernel,
    mesh=_mesh,
    out_type=jax.ShapeDtypeStruct((N_INNER, 8, 128, 8, 128), jnp.float32),
    scratch_types=[
        pltpu.VMEM((_G_PER_W * _GROUP,), jnp.int32),
        pltpu.VMEM((_NBUF, _GROUP * 2), jnp.int32),
        pltpu.VMEM((_NBUF, _GROUP * 2, EMB_DIM // 2), jnp.float32),
        pltpu.VMEM((_NBUF, 8, _TILES, 8, 128), jnp.float32),
        pltpu.SemaphoreType.DMA,
        pltpu.SemaphoreType.DMA((_NBUF,)),
        pltpu.SemaphoreType.DMA((_NBUF,)),
    ],
    compiler_params=pltpu.CompilerParams(use_tc_tiling_on_sc=False),
)
def _gather_kernel(xt_hbm, pk_hbm, out_hbm, idx_v, idx2_v, rows_v, t_v,
                   isem, gsem, wsem):
    wid = lax.axis_index("s") * _NC + lax.axis_index("c")
    g0 = wid * _G_PER_W
    n0 = g0 * _GROUP
    nw = _G_PER_W * _GROUP
    pltpu.async_copy(xt_hbm.at[pl.ds(n0, nw)], idx_v, isem)
    pltpu.make_async_copy(xt_hbm.at[pl.ds(0, nw)], idx_v, isem).wait()

    def start_gather(s, b):
        i2 = idx2_v.at[b]
        for k in range(_GROUP // 16):
            xv = idx_v[pl.ds(s * _GROUP + k * 16, 16)]
            w0 = xv * 2
            i2[pl.ds(k * 16, 16)] = w0
            i2[pl.ds(_GROUP + k * 16, 16)] = w0 + 1
        pltpu.async_copy(
            pk_hbm.at[idx2_v.at[b]], rows_v.at[b], gsem.at[b])

    def wait_gather(b):
        pltpu.make_async_copy(
            pk_hbm.at[idx2_v.at[b]], rows_v.at[b], gsem.at[b]).wait()

    def transpose(b):
        rows = rows_v.at[b]
        tref = t_v.at[b]

        def iblock(ib, _):
            row0 = ib * 16
            for d0 in range(0, EMB_DIM, 16):
                h = d0 // 32
                c0 = d0 % 32
                regs = [rows[h * _GROUP + row0 + p, pl.ds(c0, 16)]
                        for p in range(16)]
                tregs = _transpose16(regs)
                for p in range(16):
                    d = d0 + p
                    tref[d // 8, ib // 8, d % 8,
                         pl.ds((ib % 8) * 16, 16)] = tregs[p]
            return ()

        lax.fori_loop(0, _GROUP // 16, iblock, ())

    def start_wb(s, b):
        g = g0 + s
        pltpu.async_copy(
            t_v.at[b],
            out_hbm.at[g // _TPJ, :, pl.ds((g % _TPJ) * _TILES, _TILES)],
            wsem.at[b])

    def wait_wb(b):
        pltpu.make_async_copy(
            pk_hbm.at[pl.ds(0, _GROUP * 2)], rows_v.at[b],
            wsem.at[b]).wait()

    start_gather(0, 0)

    def step(m, _):
        for u in range(_NBUF):
            s = m * _NBUF + u
            b = u
            b1 = (u + 1) % _NBUF

            @pl.when(s + 1 < _G_PER_W)
            def _():
                start_gather(s + 1, b1)

            @pl.when(s < _G_PER_W)
            def _():
                wait_gather(b)

                @pl.when(s >= _NBUF)
                def _():
                    wait_wb(b)

                transpose(b)
                start_wb(s, b)

        return ()

    nsteps = (_G_PER_W + _NBUF - 1) // _NBUF
    lax.fori_loop(0, nsteps, step, ())

    for b in range(_NBUF):
        wait_wb(b)


def kernel(x, table):
    tt = table.T
    packed = _pack_kernel(tt, lax.slice(tt, (0, NROWS - 64), (EMB_DIM, NROWS)))
    pk2 = packed.reshape(2 * NROWS, EMB_DIM // 2)
    xt = x.T.astype(jnp.int32).reshape(-1)
    out5 = _gather_kernel(xt, pk2)
    return out5.transpose(2, 4, 0, 1, 3).reshape(N_OUTER, N_INNER, EMB_DIM)
